# baseline (speedup 1.0000x reference)
import functools

import jax
import jax.numpy as jnp
from jax import lax
from jax.experimental import pallas as pl
from jax.experimental.pallas import tpu as pltpu
from jax.experimental.pallas import tpu_sc as plsc

HID = 64
_NW = 32
_CHUNK = 128
_NSLOT = 8


_CH_FAST = 264
_CH_SLOW = 88


def _sc_gather(table32, idx2, n_node_ch, n_edge_ch):
    n_chunks, ck = idx2.shape
    d = table32.shape[1]
    n_dump_ch = n_chunks - n_node_ch - n_edge_ch
    mesh = plsc.VectorSubcoreMesh(core_axis_name="c", subcore_axis_name="s")

    @functools.partial(
        pl.kernel, mesh=mesh,
        out_type=[
            jax.ShapeDtypeStruct((n_node_ch * ck, d), jnp.float32),
            jax.ShapeDtypeStruct((n_edge_ch * ck, d), jnp.float32),
            jax.ShapeDtypeStruct((n_dump_ch * ck, d), jnp.float32),
        ],
        compiler_params=pltpu.CompilerParams(use_tc_tiling_on_sc=False),
        scratch_types=[
            pltpu.VMEM((_CH_FAST, ck), jnp.int32),
            pltpu.VMEM((_NSLOT, ck, d), jnp.float32),
            pltpu.SemaphoreType.DMA((_NSLOT,)),
            pltpu.SemaphoreType.DMA((_NSLOT,)),
        ])
    def gather_kernel(table_hbm, idx_hbm, node_hbm, edge_hbm, dump_hbm,
                      idx_v, buf, gsem, ssem):
        c = lax.axis_index("c")
        s = lax.axis_index("s")

        def store_chunk(base, g, b):
            pltpu.make_async_copy(
                table_hbm.at[idx_v.at[0]], buf.at[b], gsem.at[b]).wait()
            gg = base + g
            src = buf.at[b]

            @pl.when(gg < n_node_ch)
            def _():
                pltpu.async_copy(
                    src, node_hbm.at[pl.ds(gg * ck, ck)], ssem.at[b])

            @pl.when(jnp.logical_and(gg >= n_node_ch,
                                     gg < n_node_ch + n_edge_ch))
            def _():
                pltpu.async_copy(
                    src, edge_hbm.at[pl.ds((gg - n_node_ch) * ck, ck)],
                    ssem.at[b])

            @pl.when(gg >= n_node_ch + n_edge_ch)
            def _():
                pltpu.async_copy(
                    src,
                    dump_hbm.at[pl.ds((gg - n_node_ch - n_edge_ch) * ck, ck)],
                    ssem.at[b])

        def wait_store(b):
            pltpu.make_async_copy(
                buf.at[b], node_hbm.at[pl.ds(0, ck)], ssem.at[b]).wait()

        def ring(base, nloc):
            pltpu.sync_copy(idx_hbm.at[pl.ds(base, nloc)],
                            idx_v.at[pl.ds(0, nloc)])
            for b in range(_NSLOT):
                pltpu.async_copy(
                    table_hbm.at[idx_v.at[b]], buf.at[b], gsem.at[b])

            @pl.loop(0, nloc - _NSLOT, step=_NSLOT)
            def _(g0):
                for b in range(_NSLOT):
                    g = g0 + b
                    store_chunk(base, g, b)
                    wait_store(b)
                    pltpu.async_copy(
                        table_hbm.at[idx_v.at[g + _NSLOT]], buf.at[b],
                        gsem.at[b])

            for b in range(_NSLOT):
                store_chunk(base, nloc - _NSLOT + b, b)
            for b in range(_NSLOT):
                wait_store(b)

        @pl.when(c == 0)
        def _():
            ring(s * _CH_FAST, _CH_FAST)

        @pl.when(c == 1)
        def _():
            ring(16 * _CH_FAST + s * _CH_SLOW, _CH_SLOW)

    return gather_kernel(table32, idx2)


def _acc_mm_body(x_ref, we_ref, wo_ref, o_ref):
    g = pl.program_id(0)
    x = x_ref[...]
    m = x.shape[0]
    xb = pltpu.bitcast(x, jnp.bfloat16).reshape(m, 2, 128)
    acc = (jnp.dot(xb[:, 0, :], we_ref[0], preferred_element_type=jnp.float32)
           + jnp.dot(xb[:, 1, :], wo_ref[0],
                     preferred_element_type=jnp.float32))

    @pl.when(g == 0)
    def _():
        o_ref[...] = jnp.zeros_like(o_ref)

    o_ref[...] += acc


def _grouped_matmul(rows, we3, wo3, n_rows):
    ng = we3.shape[0]
    return pl.pallas_call(
        _acc_mm_body,
        grid=(ng,),
        in_specs=[pl.BlockSpec((n_rows, 128), lambda g: (g, 0)),
                  pl.BlockSpec((1, 128, HID), lambda g: (g, 0, 0)),
                  pl.BlockSpec((1, 128, HID), lambda g: (g, 0, 0))],
        out_specs=pl.BlockSpec((n_rows, HID), lambda g: (0, 0)),
        out_shape=jax.ShapeDtypeStruct((n_rows, HID), jnp.float32),
    )(rows, we3, wo3)


def _split_weight(w, seq_len):
    wr = w.reshape(seq_len // 4, 4, HID, HID)
    we = wr[:, :, 0::2, :].reshape(-1, 128, HID).astype(jnp.bfloat16)
    wo = wr[:, :, 1::2, :].reshape(-1, 128, HID).astype(jnp.bfloat16)
    return we, wo


def _h2u_body(h_ref, agg_ref, wt_ref, u_ref):
    h2 = jnp.maximum(h_ref[...] + agg_ref[...], 0.0)
    u_ref[...] = jnp.dot(h2, wt_ref[...], preferred_element_type=jnp.float32)


def _tail_body(z_ref, bt_ref, wlm_ref, blm_ref, et_ref, lab_ref, probs_ref):
    et = et_ref[...]
    special = et <= 3
    masked = ((et * 131071) % 100 < 15) & (~special)
    lab_ref[...] = jnp.where(masked, et, -100)
    s = jnp.tanh(z_ref[...] + bt_ref[...][None, :])
    logits = jnp.dot(s, wlm_ref[...], preferred_element_type=jnp.float32)
    logits = logits + blm_ref[...][None, :]
    m = jnp.max(logits, axis=-1, keepdims=True)
    p = jnp.exp(logits - m)
    p = p / jnp.sum(p, axis=-1, keepdims=True)
    probs_ref[...] = p[:, None, :]


def kernel(node_tokens, edge_tokens, edge_index, emb_table, W_gnn, W_edge,
           W_t, b_t, W_lm, b_lm):
    n_nodes, l_node = node_tokens.shape
    n_edges, l_edge = edge_tokens.shape
    vocab = W_lm.shape[1]
    src, dst = edge_index[0], edge_index[1]

    tab16 = emb_table.astype(jnp.bfloat16)
    tab32 = lax.bitcast_convert_type(
        tab16.reshape(emb_table.shape[0], HID // 2, 2), jnp.float32)

    idx_n = node_tokens.reshape(n_nodes, l_node // 4, 4
                                ).transpose(1, 0, 2).reshape(-1)
    idx_e = edge_tokens.reshape(n_edges, l_edge // 4, 4
                                ).transpose(1, 0, 2).reshape(-1)
    n_node_ch = idx_n.size // _CHUNK
    n_edge_ch = idx_e.size // _CHUNK
    tot = idx_n.size + idx_e.size
    nch = -(-tot // (_NW * _CHUNK))
    nch = -(-nch // _NSLOT) * _NSLOT
    pad = _NW * nch * _CHUNK - tot
    flat = jnp.concatenate([idx_n, idx_e, jnp.zeros((pad,), idx_n.dtype)])
    rows_n, rows_e, _ = _sc_gather(
        tab32, flat.reshape(-1, _CHUNK), n_node_ch, n_edge_ch)

    we_n, wo_n = _split_weight(W_gnn, l_node)
    we_e, wo_e = _split_weight(W_edge, l_edge)
    h = _grouped_matmul(rows_n.reshape(-1, 128), we_n, wo_n, n_nodes)
    msg_e = _grouped_matmul(rows_e.reshape(-1, 128), we_e, wo_e, n_edges)

    msg = h[src] + msg_e
    agg = jnp.zeros_like(h).at[dst].add(msg)

    u = pl.pallas_call(
        _h2u_body,
        grid=(1,),
        in_specs=[pl.BlockSpec((n_nodes, HID), lambda i: (0, 0)),
                  pl.BlockSpec((n_nodes, HID), lambda i: (0, 0)),
                  pl.BlockSpec((HID, HID), lambda i: (0, 0))],
        out_specs=pl.BlockSpec((n_nodes, HID), lambda i: (0, 0)),
        out_shape=jax.ShapeDtypeStruct((n_nodes, HID), jnp.float32),
    )(h, agg, W_t)

    z = u[src] + u[dst]

    rb = 200
    labels, probs = pl.pallas_call(
        _tail_body,
        grid=(n_edges // rb,),
        in_specs=[pl.BlockSpec((rb, HID), lambda i: (i, 0)),
                  pl.BlockSpec((HID,), lambda i: (0,)),
                  pl.BlockSpec((HID, vocab), lambda i: (0, 0)),
                  pl.BlockSpec((vocab,), lambda i: (0,)),
                  pl.BlockSpec((rb, l_edge), lambda i: (i, 0))],
        out_specs=[pl.BlockSpec((rb, l_edge), lambda i: (i, 0)),
                   pl.BlockSpec((rb, 1, vocab), lambda i: (i, 0, 0))],
        out_shape=[jax.ShapeDtypeStruct((n_edges, l_edge), jnp.int32),
                   jax.ShapeDtypeStruct((n_edges, 1, vocab), jnp.float32)],
    )(z, b_t, W_lm, b_lm, edge_tokens)

    return (labels, probs)


# speedup vs baseline: 1.0337x; 1.0337x over previous
import functools

import jax
import jax.numpy as jnp
from jax import lax
from jax.experimental import pallas as pl
from jax.experimental.pallas import tpu as pltpu
from jax.experimental.pallas import tpu_sc as plsc

HID = 64
_NW = 32
_CHUNK = 128
_NSLOT = 8


_CH_FAST = 176
_CH_SLOW = 176


def _sc_gather(table32, idx2, n_node_ch, n_edge_ch):
    n_chunks, ck = idx2.shape
    d = table32.shape[1]
    n_dump_ch = n_chunks - n_node_ch - n_edge_ch
    mesh = plsc.VectorSubcoreMesh(core_axis_name="c", subcore_axis_name="s")

    @functools.partial(
        pl.kernel, mesh=mesh,
        out_type=[
            jax.ShapeDtypeStruct((n_node_ch * ck, d), jnp.float32),
            jax.ShapeDtypeStruct((n_edge_ch * ck, d), jnp.float32),
            jax.ShapeDtypeStruct((n_dump_ch * ck, d), jnp.float32),
        ],
        compiler_params=pltpu.CompilerParams(use_tc_tiling_on_sc=False),
        scratch_types=[
            pltpu.VMEM((_CH_FAST, ck), jnp.int32),
            pltpu.VMEM((_NSLOT, ck, d), jnp.float32),
            pltpu.SemaphoreType.DMA((_NSLOT,)),
            pltpu.SemaphoreType.DMA((_NSLOT,)),
        ])
    def gather_kernel(table_hbm, idx_hbm, node_hbm, edge_hbm, dump_hbm,
                      idx_v, buf, gsem, ssem):
        c = lax.axis_index("c")
        s = lax.axis_index("s")

        def store_chunk(base, g, b):
            pltpu.make_async_copy(
                table_hbm.at[idx_v.at[0]], buf.at[b], gsem.at[b]).wait()
            gg = base + g
            src = buf.at[b]

            @pl.when(gg < n_node_ch)
            def _():
                pltpu.async_copy(
                    src, node_hbm.at[pl.ds(gg * ck, ck)], ssem.at[b])

            @pl.when(jnp.logical_and(gg >= n_node_ch,
                                     gg < n_node_ch + n_edge_ch))
            def _():
                pltpu.async_copy(
                    src, edge_hbm.at[pl.ds((gg - n_node_ch) * ck, ck)],
                    ssem.at[b])

            @pl.when(gg >= n_node_ch + n_edge_ch)
            def _():
                pltpu.async_copy(
                    src,
                    dump_hbm.at[pl.ds((gg - n_node_ch - n_edge_ch) * ck, ck)],
                    ssem.at[b])

        def wait_store(b):
            pltpu.make_async_copy(
                buf.at[b], node_hbm.at[pl.ds(0, ck)], ssem.at[b]).wait()

        def ring(base, nloc):
            pltpu.sync_copy(idx_hbm.at[pl.ds(base, nloc)],
                            idx_v.at[pl.ds(0, nloc)])
            for b in range(_NSLOT):
                pltpu.async_copy(
                    table_hbm.at[idx_v.at[b]], buf.at[b], gsem.at[b])

            @pl.loop(0, nloc - _NSLOT, step=_NSLOT)
            def _(g0):
                for b in range(_NSLOT):
                    g = g0 + b
                    store_chunk(base, g, b)
                    wait_store(b)
                    pltpu.async_copy(
                        table_hbm.at[idx_v.at[g + _NSLOT]], buf.at[b],
                        gsem.at[b])

            for b in range(_NSLOT):
                store_chunk(base, nloc - _NSLOT + b, b)
            for b in range(_NSLOT):
                wait_store(b)

        @pl.when(c == 0)
        def _():
            ring(s * _CH_FAST, _CH_FAST)

        @pl.when(c == 1)
        def _():
            ring(16 * _CH_FAST + s * _CH_SLOW, _CH_SLOW)

    return gather_kernel(table32, idx2)


def _sc_scatter_agg(h, msg_e, src_p, dst_p, zeros):
    ne = src_p.shape[0]
    n_agg = zeros.shape[0]
    per_tile = ne // 16
    rows_t = n_agg // 16
    mesh = plsc.VectorSubcoreMesh(core_axis_name="c", subcore_axis_name="s")

    @functools.partial(
        pl.kernel, mesh=mesh,
        out_type=jax.ShapeDtypeStruct((n_agg, HID), jnp.float32),
        compiler_params=pltpu.CompilerParams(use_tc_tiling_on_sc=False),
        scratch_types=[
            pltpu.VMEM((per_tile // 128, 128), jnp.int32),
            pltpu.VMEM((per_tile // 128, 128), jnp.int32),
            pltpu.VMEM((128, HID), jnp.float32),
            pltpu.VMEM((128, HID), jnp.float32),
            pltpu.VMEM_SHARED((n_agg, HID), jnp.float32),
            pltpu.SemaphoreType.DMA,
        ])
    def scatter_kernel(h_hbm, me_hbm, src_hbm, dst_hbm, zer_hbm, out_hbm,
                       srcv, dstv, hbuf, mbuf, aggsh, sem):
        c = lax.axis_index("c")
        s = lax.axis_index("s")

        @pl.when(c == 0)
        def _():
            rbase = s * rows_t
            pltpu.sync_copy(zer_hbm.at[pl.ds(rbase, rows_t)],
                            aggsh.at[pl.ds(rbase, rows_t)])
            plsc.subcore_barrier()
            ebase = s * per_tile
            for j in range(per_tile // 128):
                pltpu.sync_copy(src_hbm.at[pl.ds(ebase + j * 128, 128)],
                                srcv.at[j])
                pltpu.sync_copy(dst_hbm.at[pl.ds(ebase + j * 128, 128)],
                                dstv.at[j])
                pltpu.async_copy(h_hbm.at[srcv.at[j]], hbuf, sem).wait()
                pltpu.sync_copy(
                    me_hbm.at[pl.ds(ebase + j * 128, 128)], mbuf)
                pltpu.sync_copy(hbuf, aggsh.at[dstv.at[j]], add=True)
                pltpu.sync_copy(mbuf, aggsh.at[dstv.at[j]], add=True)
            plsc.subcore_barrier()
            pltpu.sync_copy(aggsh.at[pl.ds(rbase, rows_t)],
                            out_hbm.at[pl.ds(rbase, rows_t)])

    return scatter_kernel(h, msg_e, src_p, dst_p, zeros)


def _acc_mm_body(x_ref, we_ref, wo_ref, o_ref):
    g = pl.program_id(0)
    x = x_ref[...]
    m = x.shape[0]
    xb = pltpu.bitcast(x, jnp.bfloat16).reshape(m, 2, 128)
    acc = (jnp.dot(xb[:, 0, :], we_ref[0], preferred_element_type=jnp.float32)
           + jnp.dot(xb[:, 1, :], wo_ref[0],
                     preferred_element_type=jnp.float32))

    @pl.when(g == 0)
    def _():
        o_ref[...] = jnp.zeros_like(o_ref)

    o_ref[...] += acc


def _grouped_matmul(rows, we3, wo3, n_rows):
    ng = we3.shape[0]
    return pl.pallas_call(
        _acc_mm_body,
        grid=(ng,),
        in_specs=[pl.BlockSpec((n_rows, 128), lambda g: (g, 0)),
                  pl.BlockSpec((1, 128, HID), lambda g: (g, 0, 0)),
                  pl.BlockSpec((1, 128, HID), lambda g: (g, 0, 0))],
        out_specs=pl.BlockSpec((n_rows, HID), lambda g: (0, 0)),
        out_shape=jax.ShapeDtypeStruct((n_rows, HID), jnp.float32),
    )(rows, we3, wo3)


def _split_weight(w, seq_len):
    wr = w.reshape(seq_len // 4, 4, HID, HID)
    we = wr[:, :, 0::2, :].reshape(-1, 128, HID).astype(jnp.bfloat16)
    wo = wr[:, :, 1::2, :].reshape(-1, 128, HID).astype(jnp.bfloat16)
    return we, wo


def _h2u_body(h_ref, agg_ref, wt_ref, u_ref):
    h2 = jnp.maximum(h_ref[...] + agg_ref[...], 0.0)
    u_ref[...] = jnp.dot(h2, wt_ref[...], preferred_element_type=jnp.float32)


def _tail_body(z_ref, bt_ref, wlm_ref, blm_ref, et_ref, lab_ref, probs_ref):
    et = et_ref[...]
    special = et <= 3
    masked = ((et * 131071) % 100 < 15) & (~special)
    lab_ref[...] = jnp.where(masked, et, -100)
    s = jnp.tanh(z_ref[...] + bt_ref[...][None, :])
    logits = jnp.dot(s, wlm_ref[...], preferred_element_type=jnp.float32)
    logits = logits + blm_ref[...][None, :]
    m = jnp.max(logits, axis=-1, keepdims=True)
    p = jnp.exp(logits - m)
    p = p / jnp.sum(p, axis=-1, keepdims=True)
    probs_ref[...] = p[:, None, :]


def kernel(node_tokens, edge_tokens, edge_index, emb_table, W_gnn, W_edge,
           W_t, b_t, W_lm, b_lm):
    n_nodes, l_node = node_tokens.shape
    n_edges, l_edge = edge_tokens.shape
    vocab = W_lm.shape[1]
    src, dst = edge_index[0], edge_index[1]

    tab16 = emb_table.astype(jnp.bfloat16)
    tab32 = lax.bitcast_convert_type(
        tab16.reshape(emb_table.shape[0], HID // 2, 2), jnp.float32)

    idx_n = node_tokens.reshape(n_nodes, l_node // 4, 4
                                ).transpose(1, 0, 2).reshape(-1)
    idx_e = edge_tokens.reshape(n_edges, l_edge // 4, 4
                                ).transpose(1, 0, 2).reshape(-1)
    n_node_ch = idx_n.size // _CHUNK
    n_edge_ch = idx_e.size // _CHUNK
    tot = idx_n.size + idx_e.size
    nch = -(-tot // (_NW * _CHUNK))
    nch = -(-nch // _NSLOT) * _NSLOT
    pad = _NW * nch * _CHUNK - tot
    flat = jnp.concatenate([idx_n, idx_e, jnp.zeros((pad,), idx_n.dtype)])
    rows_n, rows_e, _ = _sc_gather(
        tab32, flat.reshape(-1, _CHUNK), n_node_ch, n_edge_ch)

    we_n, wo_n = _split_weight(W_gnn, l_node)
    we_e, wo_e = _split_weight(W_edge, l_edge)
    h = _grouped_matmul(rows_n.reshape(-1, 128), we_n, wo_n, n_nodes)
    msg_e = _grouped_matmul(rows_e.reshape(-1, 128), we_e, wo_e, n_edges)

    ne_pad = 4096
    n_agg = 5120
    src_p = jnp.concatenate(
        [src, jnp.zeros((ne_pad - n_edges,), src.dtype)])
    dst_p = jnp.concatenate(
        [dst, jnp.full((ne_pad - n_edges,), n_nodes, dst.dtype)])
    me_p = jnp.concatenate(
        [msg_e, jnp.zeros((ne_pad - n_edges, HID), msg_e.dtype)])
    agg = _sc_scatter_agg(h, me_p, src_p, dst_p,
                          jnp.zeros((n_agg, HID), jnp.float32))

    u = pl.pallas_call(
        _h2u_body,
        grid=(1,),
        in_specs=[pl.BlockSpec((n_nodes, HID), lambda i: (0, 0)),
                  pl.BlockSpec((n_nodes, HID), lambda i: (0, 0)),
                  pl.BlockSpec((HID, HID), lambda i: (0, 0))],
        out_specs=pl.BlockSpec((n_nodes, HID), lambda i: (0, 0)),
        out_shape=jax.ShapeDtypeStruct((n_nodes, HID), jnp.float32),
    )(h, agg[:n_nodes], W_t)

    z = u[src] + u[dst]

    rb = 200
    labels, probs = pl.pallas_call(
        _tail_body,
        grid=(n_edges // rb,),
        in_specs=[pl.BlockSpec((rb, HID), lambda i: (i, 0)),
                  pl.BlockSpec((HID,), lambda i: (0,)),
                  pl.BlockSpec((HID, vocab), lambda i: (0, 0)),
                  pl.BlockSpec((vocab,), lambda i: (0,)),
                  pl.BlockSpec((rb, l_edge), lambda i: (i, 0))],
        out_specs=[pl.BlockSpec((rb, l_edge), lambda i: (i, 0)),
                   pl.BlockSpec((rb, 1, vocab), lambda i: (i, 0, 0))],
        out_shape=[jax.ShapeDtypeStruct((n_edges, l_edge), jnp.int32),
                   jax.ShapeDtypeStruct((n_edges, 1, vocab), jnp.float32)],
    )(z, b_t, W_lm, b_lm, edge_tokens)

    return (labels, probs)
